# Initial kernel scaffold; baseline (speedup 1.0000x reference)
import jax
import jax.numpy as jnp
from jax.experimental import pallas as pl

N_FEAT = 64
N_CENT = 1024
TOK = 1024


def _vq_body(c_ref, x_ref, out_ref, loss_ref):
    b = pl.program_id(0)
    C = c_ref[...]
    X = x_ref[0]
    cn = jnp.sum(C * C, axis=0)
    s = jax.lax.dot_general(X, C, (((0,), (0,)), ((), ())),
                            preferred_element_type=jnp.float32)
    neg = 2.0 * s - cn[None, :]
    idx = jnp.argmax(neg, axis=1)
    maxv = jnp.max(neg, axis=1)
    onehot = (jax.lax.broadcasted_iota(jnp.int32, (N_CENT, TOK), 0)
              == idx[None, :]).astype(jnp.float32)
    Q = jnp.dot(C, onehot, preferred_element_type=jnp.float32)
    out_ref[0] = X + (Q - X)
    xn = jnp.sum(X * X, axis=0)
    sq = jnp.sum(xn - maxv)

    @pl.when(b == 0)
    def _():
        loss_ref[0, 0] = 0.0

    loss_ref[0, 0] += sq


def kernel(x, centroids):
    B = x.shape[0]
    xr = x.reshape(B, N_FEAT, TOK)
    out, loss = pl.pallas_call(
        _vq_body,
        grid=(B,),
        in_specs=[
            pl.BlockSpec((N_FEAT, N_CENT), lambda b: (0, 0)),
            pl.BlockSpec((1, N_FEAT, TOK), lambda b: (b, 0, 0)),
        ],
        out_specs=[
            pl.BlockSpec((1, N_FEAT, TOK), lambda b: (b, 0, 0)),
            pl.BlockSpec((1, 1), lambda b: (0, 0)),
        ],
        out_shape=[
            jax.ShapeDtypeStruct((B, N_FEAT, TOK), jnp.float32),
            jax.ShapeDtypeStruct((1, 1), jnp.float32),
        ],
    )(centroids, xr)
    x_quant = out.reshape(x.shape)
    cent_loss = loss[0, 0] / x.size
    return (x_quant, cent_loss)


# speedup vs baseline: 2.3168x; 2.3168x over previous
import jax
import jax.numpy as jnp
from jax.experimental import pallas as pl

N_FEAT = 64
N_CENT = 1024
TOK = 1024


def _vq_body(c_ref, x_ref, out_ref, loss_ref):
    b = pl.program_id(0)
    C = c_ref[...]
    X = x_ref[0]
    cn = jnp.sum(C * C, axis=0)
    s = jax.lax.dot_general(X, C, (((0,), (0,)), ((), ())),
                            preferred_element_type=jnp.float32)
    neg = 2.0 * s - cn[None, :]
    idx = jnp.argmax(neg, axis=1)
    maxv = jnp.max(neg, axis=1)
    onehot = (jax.lax.broadcasted_iota(jnp.int32, (N_CENT, TOK), 0)
              == idx[None, :]).astype(jnp.float32)
    Q = jnp.dot(C, onehot, preferred_element_type=jnp.float32)
    out_ref[0] = X + (Q - X)
    xn = jnp.sum(X * X, axis=0)
    sq = jnp.sum(xn - maxv)

    @pl.when(b == 0)
    def _():
        loss_ref[...] = jnp.zeros_like(loss_ref)

    loss_ref[...] = loss_ref[...] + sq


def kernel(x, centroids):
    B = x.shape[0]
    xr = x.reshape(B, N_FEAT, TOK)
    out, loss = pl.pallas_call(
        _vq_body,
        grid=(B,),
        in_specs=[
            pl.BlockSpec((N_FEAT, N_CENT), lambda b: (0, 0)),
            pl.BlockSpec((1, N_FEAT, TOK), lambda b: (b, 0, 0)),
        ],
        out_specs=[
            pl.BlockSpec((1, N_FEAT, TOK), lambda b: (b, 0, 0)),
            pl.BlockSpec((1, 1), lambda b: (0, 0)),
        ],
        out_shape=[
            jax.ShapeDtypeStruct((B, N_FEAT, TOK), jnp.float32),
            jax.ShapeDtypeStruct((1, 1), jnp.float32),
        ],
    )(centroids, xr)
    x_quant = out.reshape(x.shape)
    cent_loss = loss[0, 0] / x.size
    return (x_quant, cent_loss)
